# Initial kernel scaffold; baseline (speedup 1.0000x reference)
import jax
import jax.numpy as jnp
from jax.experimental import pallas as pl


def kernel(idx, table):
    raise NotImplementedError("write your pallas kernel here")


# speedup vs baseline: 1.0329x; 1.0329x over previous
import functools

import jax
import jax.numpy as jnp
from jax import lax
from jax.experimental import pallas as pl
from jax.experimental.pallas import tpu as pltpu
from jax.experimental.pallas import tpu_sc as plsc

_D = 1000
_NC = 2
_NS = 16
_NW = _NC * _NS
_CHUNK = 64


@functools.lru_cache(maxsize=None)
def _build(n_rows):
  b_per_w = n_rows // _NW
  n_chunks = b_per_w // _CHUNK
  assert b_per_w * _NW == n_rows and n_chunks * _CHUNK == b_per_w

  mesh = plsc.VectorSubcoreMesh(
      core_axis_name="c", subcore_axis_name="s",
      num_cores=_NC, num_subcores=_NS)

  @functools.partial(
      pl.kernel,
      out_type=jax.ShapeDtypeStruct((n_rows, _D), jnp.float32),
      mesh=mesh,
      compiler_params=pltpu.CompilerParams(use_tc_tiling_on_sc=False),
      scratch_types=[
          pltpu.VMEM((b_per_w,), jnp.int32),
          pltpu.VMEM((_CHUNK, _D), jnp.float32),
          pltpu.VMEM((_CHUNK, _D), jnp.float32),
          pltpu.SemaphoreType.DMA,
          pltpu.SemaphoreType.DMA,
          pltpu.SemaphoreType.DMA,
          pltpu.SemaphoreType.DMA,
      ],
  )
  def emb(idx_hbm, table_hbm, out_hbm,
          idx_v, buf0, buf1, gs0, gs1, ss0, ss1):
    wid = lax.axis_index("s") * _NC + lax.axis_index("c")
    base = wid * b_per_w
    pltpu.sync_copy(idx_hbm.at[pl.ds(base, b_per_w)], idx_v)

    bufs = ((buf0, gs0, ss0), (buf1, gs1, ss1))

    def gather(c, buf, gsem):
      return pltpu.make_async_copy(
          table_hbm.at[idx_v.at[pl.ds(c * _CHUNK, _CHUNK)]], buf, gsem)

    def scatter(c, buf, ssem):
      return pltpu.make_async_copy(
          buf, out_hbm.at[pl.ds(base + c * _CHUNK, _CHUNK)], ssem)

    gather(0, buf0, gs0).start()

    @pl.loop(0, n_chunks, step=2)
    def _pair(j):
      for p in range(2):
        buf, gsem, ssem = bufs[p]
        c = j + p

        @pl.when(c < n_chunks)
        def _chunk():
          nxt = c + 1

          @pl.when(nxt < n_chunks)
          def _start_next():
            obuf, ogsem, ossem = bufs[1 - p]

            @pl.when(nxt >= 2)
            def _wait_prev_scatter():
              scatter(nxt - 2, obuf, ossem).wait()
            gather(nxt, obuf, ogsem).start()

          gather(c, buf, gsem).wait()
          scatter(c, buf, ssem).start()

    @pl.when(n_chunks >= 2)
    def _drain_a():
      b, _, ssem = bufs[(n_chunks - 2) % 2]
      scatter(n_chunks - 2, b, ssem).wait()
    b_last, _, ssem_last = bufs[(n_chunks - 1) % 2]
    scatter(n_chunks - 1, b_last, ssem_last).wait()

  return emb


@jax.jit
def kernel(idx, table):
  b, t = idx.shape
  flat = idx.astype(jnp.int32).reshape(b * t)
  out = _build(b * t)(flat, table)
  return out.reshape(b, t, _D)


# speedup vs baseline: 1.0360x; 1.0030x over previous
import functools

import jax
import jax.numpy as jnp
from jax import lax
from jax.experimental import pallas as pl
from jax.experimental.pallas import tpu as pltpu
from jax.experimental.pallas import tpu_sc as plsc

_D = 1000
_NC = 2
_NS = 16
_NW = _NC * _NS
_TP = 56


@functools.lru_cache(maxsize=None)
def _build(nb, t):
  b_per_w = nb // _NW
  assert b_per_w * _NW == nb

  mesh = plsc.VectorSubcoreMesh(
      core_axis_name="c", subcore_axis_name="s",
      num_cores=_NC, num_subcores=_NS)

  @functools.partial(
      pl.kernel,
      out_type=jax.ShapeDtypeStruct((nb, t, _D), jnp.float32),
      mesh=mesh,
      compiler_params=pltpu.CompilerParams(use_tc_tiling_on_sc=False),
      scratch_types=[
          pltpu.VMEM((b_per_w * _TP,), jnp.int32),
          pltpu.VMEM((t, _D), jnp.float32),
          pltpu.VMEM((t, _D), jnp.float32),
          pltpu.SemaphoreType.DMA,
          pltpu.SemaphoreType.DMA,
          pltpu.SemaphoreType.DMA,
          pltpu.SemaphoreType.DMA,
      ],
  )
  def emb(idx_hbm, table_hbm, out_hbm,
          idx_v, buf0, buf1, gs0, gs1, ss0, ss1):
    wid = lax.axis_index("s") * _NC + lax.axis_index("c")
    b0 = wid * b_per_w
    pltpu.sync_copy(idx_hbm.at[pl.ds(b0 * _TP, b_per_w * _TP)], idx_v)

    bufs = ((buf0, gs0, ss0), (buf1, gs1, ss1))

    def gather(c, buf, gsem):
      return pltpu.make_async_copy(
          table_hbm.at[idx_v.at[pl.ds(c * _TP, t)]], buf, gsem)

    def scatter(c, buf, ssem):
      return pltpu.make_async_copy(buf, out_hbm.at[b0 + c], ssem)

    gather(0, buf0, gs0).start()

    @pl.loop(0, b_per_w, step=2)
    def _pair(j):
      for p in range(2):
        buf, gsem, ssem = bufs[p]
        c = j + p
        nxt = c + 1

        @pl.when(nxt < b_per_w)
        def _start_next():
          obuf, ogsem, ossem = bufs[1 - p]

          @pl.when(nxt >= 2)
          def _wait_prev_scatter():
            scatter(nxt - 2, obuf, ossem).wait()
          gather(nxt, obuf, ogsem).start()

        gather(c, buf, gsem).wait()
        scatter(c, buf, ssem).start()

    scatter(b_per_w - 2, buf0, ss0).wait()
    scatter(b_per_w - 1, buf1, ss1).wait()

  return emb


@jax.jit
def kernel(idx, table):
  b, t = idx.shape
  idx_p = jnp.pad(idx.astype(jnp.int32), ((0, 0), (0, _TP - t))).reshape(-1)
  return _build(b, t)(idx_p, table)
